# baseline (speedup 1.0000x reference)
import jax
import jax.numpy as jnp
from jax.experimental import pallas as pl
from jax.experimental.pallas import tpu as pltpu

_LEAK = 0.1
_LTILE = 65536
_SUBLANE = 8


def _fused_mlp_kernel(x_ref, w0_ref, w1_ref, w2_ref, o_ref):
    bf = jnp.bfloat16
    leak = jnp.bfloat16(_LEAK)
    h = jnp.dot(w0_ref[...].astype(bf), x_ref[...],
                preferred_element_type=jnp.float32).astype(bf)
    h = jnp.maximum(h, leak * h)
    h = jnp.dot(w1_ref[...].astype(bf), h,
                preferred_element_type=jnp.float32).astype(bf)
    h = jnp.maximum(h, leak * h)
    w2col = w2_ref[...].astype(bf).reshape(64, 1)
    o_ref[...] = jnp.sum((h * w2col).astype(jnp.float32), axis=0,
                         keepdims=True)


def _round_up(n, m):
    return -(-n // m) * m


def kernel(x_nchw, w0, w1, w2):
    B, c_in = x_nchw.shape[0], x_nchw.shape[1]
    c_out = w2.shape[0]

    xt = x_nchw.reshape(B, c_in).astype(jnp.bfloat16).T
    w0m = w0.reshape(w0.shape[0], c_in)
    w1m = w1.reshape(w1.shape[0], w1.shape[1])
    w2m = w2.reshape(c_out, w2.shape[1])

    ltile = min(_LTILE, _round_up(B, 128))
    bpad = _round_up(B, ltile)
    if bpad != B:
        xt = jnp.pad(xt, ((0, 0), (0, bpad - B)))
    grid = (bpad // ltile,)

    out = pl.pallas_call(
        _fused_mlp_kernel,
        out_shape=jax.ShapeDtypeStruct((c_out, bpad), jnp.float32),
        grid=grid,
        in_specs=[
            pl.BlockSpec((c_in, ltile), lambda i: (0, i)),
            pl.BlockSpec(w0m.shape, lambda i: (0, 0)),
            pl.BlockSpec(w1m.shape, lambda i: (0, 0)),
            pl.BlockSpec(w2m.shape, lambda i: (0, 0)),
        ],
        out_specs=pl.BlockSpec((c_out, ltile), lambda i: (0, i)),
        compiler_params=pltpu.CompilerParams(
            dimension_semantics=("parallel",),
            vmem_limit_bytes=64 * 1024 * 1024,
        ),
    )(xt, w0m, w1m, w2m)

    return out[:, :B].reshape(B, c_out, 1, 1)


# speedup vs baseline: 1.0137x; 1.0137x over previous
import jax
import jax.numpy as jnp
from jax.experimental import pallas as pl
from jax.experimental.pallas import tpu as pltpu

_LEAK = 0.1
_LTILE = 65536
_SUBLANE = 8


def _fused_mlp_kernel(x_ref, w0_ref, w1_ref, w2_ref, o_ref):
    bf = jnp.bfloat16
    leak = jnp.bfloat16(_LEAK)
    h = jnp.dot(w0_ref[...].astype(bf), x_ref[...],
                preferred_element_type=jnp.float32).astype(bf)
    h = jnp.maximum(h, leak * h)
    h = jnp.dot(w1_ref[...].astype(bf), h,
                preferred_element_type=jnp.float32).astype(bf)
    h = jnp.maximum(h, leak * h)
    o_ref[...] = jnp.dot(w2_ref[...].astype(bf), h,
                         preferred_element_type=jnp.float32)


def _round_up(n, m):
    return -(-n // m) * m


def kernel(x_nchw, w0, w1, w2):
    B, c_in = x_nchw.shape[0], x_nchw.shape[1]
    c_out = w2.shape[0]

    xt = x_nchw.reshape(B, c_in).astype(jnp.bfloat16).T
    w0m = w0.reshape(w0.shape[0], c_in)
    w1m = w1.reshape(w1.shape[0], w1.shape[1])
    w2m = w2.reshape(c_out, w2.shape[1])

    ltile = min(_LTILE, _round_up(B, 128))
    bpad = _round_up(B, ltile)
    if bpad != B:
        xt = jnp.pad(xt, ((0, 0), (0, bpad - B)))
    grid = (bpad // ltile,)

    out = pl.pallas_call(
        _fused_mlp_kernel,
        out_shape=jax.ShapeDtypeStruct((c_out, bpad), jnp.float32),
        grid=grid,
        in_specs=[
            pl.BlockSpec((c_in, ltile), lambda i: (0, i)),
            pl.BlockSpec(w0m.shape, lambda i: (0, 0)),
            pl.BlockSpec(w1m.shape, lambda i: (0, 0)),
            pl.BlockSpec(w2m.shape, lambda i: (0, 0)),
        ],
        out_specs=pl.BlockSpec((c_out, ltile), lambda i: (0, i)),
        compiler_params=pltpu.CompilerParams(
            dimension_semantics=("parallel",),
            vmem_limit_bytes=64 * 1024 * 1024,
        ),
    )(xt, w0m, w1m, w2m)

    return out[:, :B].reshape(B, c_out, 1, 1)


# speedup vs baseline: 1.0147x; 1.0009x over previous
import jax
import jax.numpy as jnp
from jax.experimental import pallas as pl
from jax.experimental.pallas import tpu as pltpu

_LEAK = 0.1
_LTILE = 32768
_SUBLANE = 8


def _fused_mlp_kernel(x_ref, w0_ref, w1_ref, w2_ref, o_ref):
    bf = jnp.bfloat16
    leak = jnp.bfloat16(_LEAK)
    h = jnp.dot(w0_ref[...].astype(bf), x_ref[...],
                preferred_element_type=jnp.float32).astype(bf)
    h = jnp.maximum(h, leak * h)
    h = jnp.dot(w1_ref[...].astype(bf), h,
                preferred_element_type=jnp.float32).astype(bf)
    h = jnp.maximum(h, leak * h)
    o_ref[...] = jnp.dot(w2_ref[...].astype(bf), h,
                         preferred_element_type=jnp.float32)


def _round_up(n, m):
    return -(-n // m) * m


def kernel(x_nchw, w0, w1, w2):
    B, c_in = x_nchw.shape[0], x_nchw.shape[1]
    c_out = w2.shape[0]

    xt = x_nchw.reshape(B, c_in).astype(jnp.bfloat16).T
    w0m = w0.reshape(w0.shape[0], c_in)
    w1m = w1.reshape(w1.shape[0], w1.shape[1])
    w2m = w2.reshape(c_out, w2.shape[1])

    ltile = min(_LTILE, _round_up(B, 128))
    bpad = _round_up(B, ltile)
    if bpad != B:
        xt = jnp.pad(xt, ((0, 0), (0, bpad - B)))
    grid = (bpad // ltile,)

    out = pl.pallas_call(
        _fused_mlp_kernel,
        out_shape=jax.ShapeDtypeStruct((c_out, bpad), jnp.float32),
        grid=grid,
        in_specs=[
            pl.BlockSpec((c_in, ltile), lambda i: (0, i)),
            pl.BlockSpec(w0m.shape, lambda i: (0, 0)),
            pl.BlockSpec(w1m.shape, lambda i: (0, 0)),
            pl.BlockSpec(w2m.shape, lambda i: (0, 0)),
        ],
        out_specs=pl.BlockSpec((c_out, ltile), lambda i: (0, i)),
        compiler_params=pltpu.CompilerParams(
            dimension_semantics=("parallel",),
            vmem_limit_bytes=64 * 1024 * 1024,
        ),
    )(xt, w0m, w1m, w2m)

    return out[:, :B].reshape(B, c_out, 1, 1)


# speedup vs baseline: 1.0155x; 1.0008x over previous
import jax
import jax.numpy as jnp
from jax.experimental import pallas as pl
from jax.experimental.pallas import tpu as pltpu

_LEAK = 0.1
_LTILE = 32768


def _fused_mlp_kernel(x_ref, w0_ref, w1_ref, w2_ref, o_ref):
    bf = jnp.bfloat16
    leak = jnp.bfloat16(_LEAK)
    h = jnp.dot(w0_ref[...].astype(bf), x_ref[...],
                preferred_element_type=jnp.float32).astype(bf)
    h = jnp.maximum(h, leak * h)
    h = jnp.dot(w1_ref[...].astype(bf), h,
                preferred_element_type=jnp.float32).astype(bf)
    h = jnp.maximum(h, leak * h)
    o_ref[...] = jnp.dot(w2_ref[...].astype(bf), h,
                         preferred_element_type=jnp.float32)


def _round_up(n, m):
    return -(-n // m) * m


def kernel(x_nchw, w0, w1, w2):
    B, c_in = x_nchw.shape[0], x_nchw.shape[1]
    c_out = w2.shape[0]

    xt = x_nchw.reshape(B, c_in).astype(jnp.bfloat16).T
    w0m = w0.reshape(w0.shape[0], c_in)
    w1m = w1.reshape(w1.shape[0], w1.shape[1])
    w2m = w2.reshape(c_out, w2.shape[1])

    ltile = min(_LTILE, _round_up(B, 128))
    bpad = _round_up(B, ltile)
    if bpad != B:
        xt = jnp.pad(xt, ((0, 0), (0, bpad - B)))
    grid = (bpad // ltile,)

    out = pl.pallas_call(
        _fused_mlp_kernel,
        out_shape=jax.ShapeDtypeStruct((c_out, bpad), jnp.float32),
        grid=grid,
        in_specs=[
            pl.BlockSpec((c_in, ltile), lambda i: (0, i)),
            pl.BlockSpec(w0m.shape, lambda i: (0, 0)),
            pl.BlockSpec(w1m.shape, lambda i: (0, 0)),
            pl.BlockSpec(w2m.shape, lambda i: (0, 0)),
        ],
        out_specs=pl.BlockSpec((c_out, ltile), lambda i: (0, i)),
        compiler_params=pltpu.CompilerParams(
            dimension_semantics=("parallel",),
            vmem_limit_bytes=64 * 1024 * 1024,
        ),
    )(xt, w0m, w1m, w2m)

    return out[:, :B].reshape(B, c_out, 1, 1)
